# Initial kernel scaffold; baseline (speedup 1.0000x reference)
import jax
import jax.numpy as jnp
from jax.experimental import pallas as pl


def kernel(input_ids, tables):
    raise NotImplementedError("write your pallas kernel here")


# speedup vs baseline: 1.1214x; 1.1214x over previous
import functools

import jax
import jax.numpy as jnp
from jax import lax
from jax.experimental import pallas as pl
from jax.experimental.pallas import tpu as pltpu
from jax.experimental.pallas import tpu_sc as plsc

VOCAB = 100000
N_INDEX = 26
EMBED_DIM = 32
BATCH = 16384

ROWS = BATCH * N_INDEX
GROUP = 1024
SUB = 128
NSUB = GROUP // SUB


@functools.lru_cache(maxsize=1)
def _build():
    info = plsc.get_sparse_core_info()
    nc, ns, nl = info.num_cores, info.num_subcores, info.num_lanes
    nw = nc * ns
    rows_per_w = ROWS // nw
    ngroups = rows_per_w // GROUP

    mesh = plsc.VectorSubcoreMesh(core_axis_name="c", subcore_axis_name="s")

    @functools.partial(
        pl.kernel,
        mesh=mesh,
        out_type=jax.ShapeDtypeStruct((nw, ngroups, GROUP, EMBED_DIM),
                                      jnp.float32),
        compiler_params=pltpu.CompilerParams(use_tc_tiling_on_sc=False),
        scratch_types=[
            pltpu.VMEM((NSUB, SUB), jnp.int32),
            pltpu.VMEM((NSUB, SUB), jnp.int32),
            pltpu.VMEM((GROUP, EMBED_DIM), jnp.float32),
            pltpu.SemaphoreType.DMA,
        ],
    )
    def embed_kernel(ids_hbm, table_hbm, out_hbm, ids_v, idx_v, rows_v, sem):
        wid = lax.axis_index("s") * nc + lax.axis_index("c")
        viota = lax.iota(jnp.int32, nl)

        def group_body(g, carry):
            row0 = wid * rows_per_w + g * GROUP
            pltpu.sync_copy(ids_hbm.at[wid, g], ids_v)
            for j in range(NSUB):
                for t in range(SUB // nl):
                    base = row0 + (j * SUB + t * nl)
                    field = lax.rem(base + viota, N_INDEX)
                    raw = ids_v[j, pl.ds(t * nl, nl)]
                    idx_v[j, pl.ds(t * nl, nl)] = raw + field * VOCAB
            handles = [
                pltpu.async_copy(table_hbm.at[idx_v.at[j]],
                                 rows_v.at[pl.ds(j * SUB, SUB)], sem)
                for j in range(NSUB)
            ]
            for h in handles:
                h.wait()
            pltpu.sync_copy(rows_v, out_hbm.at[wid, g])
            return carry

        lax.fori_loop(0, ngroups, group_body, 0)

    return embed_kernel, nw, ngroups


def kernel(input_ids, tables):
    embed_kernel, nw, ngroups = _build()
    ids4 = input_ids.reshape(nw, ngroups, NSUB, SUB)
    table2 = tables.reshape(N_INDEX * VOCAB, EMBED_DIM)
    out = embed_kernel(ids4, table2)
    return out.reshape(BATCH, N_INDEX, EMBED_DIM)


# speedup vs baseline: 4.4540x; 3.9719x over previous
import functools

import jax
import jax.numpy as jnp
from jax import lax
from jax.experimental import pallas as pl
from jax.experimental.pallas import tpu as pltpu
from jax.experimental.pallas import tpu_sc as plsc

VOCAB = 100000
N_INDEX = 26
EMBED_DIM = 32
BATCH = 16384

CHUNK = 4096
NCHUNK = BATCH // CHUNK


@functools.lru_cache(maxsize=1)
def _build():
    info = plsc.get_sparse_core_info()
    nc, ns, nl = info.num_cores, info.num_subcores, info.num_lanes
    mesh = plsc.VectorSubcoreMesh(core_axis_name="c", subcore_axis_name="s")

    @functools.partial(
        pl.kernel,
        mesh=mesh,
        out_type=jax.ShapeDtypeStruct((N_INDEX, EMBED_DIM, BATCH),
                                      jnp.float32),
        compiler_params=pltpu.CompilerParams(needs_layout_passes=False),
        scratch_types=[
            pltpu.VMEM((VOCAB,), jnp.float32),
            pltpu.VMEM((CHUNK,), jnp.int32),
            pltpu.VMEM((CHUNK,), jnp.float32),
            pltpu.SemaphoreType.DMA,
        ],
    )
    def embed_kernel(ids_hbm, tab_hbm, out_hbm, row_v, ids_v, out_v, sem):
        d = lax.axis_index("s") * nc + lax.axis_index("c")

        def table_body(i, carry):
            pltpu.sync_copy(tab_hbm.at[i, d, :], row_v)

            def chunk_body(cc, carry2):
                b0 = cc * CHUNK
                pltpu.sync_copy(ids_hbm.at[i, pl.ds(b0, CHUNK)], ids_v)
                for k in range(CHUNK // nl):
                    idx = ids_v[pl.ds(k * nl, nl)]
                    out_v[pl.ds(k * nl, nl)] = plsc.load_gather(row_v, [idx])
                pltpu.sync_copy(out_v, out_hbm.at[i, d, pl.ds(b0, CHUNK)])
                return carry2

            lax.fori_loop(0, NCHUNK, chunk_body, 0)
            return carry

        lax.fori_loop(0, N_INDEX, table_body, 0)

    return embed_kernel


def kernel(input_ids, tables):
    embed_kernel = _build()
    ids_t = input_ids.T
    tab_t = jnp.transpose(tables, (0, 2, 1))
    out_t = embed_kernel(ids_t, tab_t)
    return jnp.transpose(out_t, (2, 0, 1))


# speedup vs baseline: 6.2624x; 1.4060x over previous
import functools

import jax
import jax.numpy as jnp
from jax import lax
from jax.experimental import pallas as pl
from jax.experimental.pallas import tpu as pltpu
from jax.experimental.pallas import tpu_sc as plsc

VOCAB = 100000
N_INDEX = 26
EMBED_DIM = 32
BATCH = 16384

CHUNK = 4096
NCHUNK = BATCH // CHUNK
UNROLL = 16


@functools.lru_cache(maxsize=1)
def _build():
    info = plsc.get_sparse_core_info()
    nc, ns, nl = info.num_cores, info.num_subcores, info.num_lanes
    mesh = plsc.VectorSubcoreMesh(core_axis_name="c", subcore_axis_name="s")

    @functools.partial(
        pl.kernel,
        mesh=mesh,
        out_type=jax.ShapeDtypeStruct((N_INDEX, EMBED_DIM, BATCH),
                                      jnp.float32),
        compiler_params=pltpu.CompilerParams(needs_layout_passes=False),
        scratch_types=[
            pltpu.VMEM((VOCAB,), jnp.float32),
            pltpu.VMEM((CHUNK,), jnp.int32),
            pltpu.VMEM((CHUNK,), jnp.int32),
            pltpu.VMEM((CHUNK,), jnp.float32),
            pltpu.VMEM((CHUNK,), jnp.float32),
            pltpu.SemaphoreType.DMA,
            pltpu.SemaphoreType.DMA,
            pltpu.SemaphoreType.DMA,
        ],
    )
    def embed_kernel(ids_hbm, tab_hbm, out_hbm,
                     row_v, ids_a, ids_b, out_a, out_b,
                     row_sem, ids_sem, out_sem):
        d = lax.axis_index("s") * nc + lax.axis_index("c")
        ids_bufs = (ids_a, ids_b)
        out_bufs = (out_a, out_b)

        pltpu.async_copy(tab_hbm.at[0, d, :], row_v, row_sem)
        pltpu.async_copy(ids_hbm.at[0, pl.ds(0, CHUNK)], ids_a, ids_sem)

        def table_body(i, carry):
            pltpu.make_async_copy(tab_hbm.at[0, d, :], row_v, row_sem).wait()
            for cc in range(NCHUNK):
                ids_v = ids_bufs[cc % 2]
                out_v = out_bufs[cc % 2]
                pltpu.make_async_copy(
                    ids_hbm.at[0, pl.ds(0, CHUNK)], ids_v, ids_sem).wait()
                if cc < NCHUNK - 1:
                    pltpu.async_copy(
                        ids_hbm.at[i, pl.ds((cc + 1) * CHUNK, CHUNK)],
                        ids_bufs[(cc + 1) % 2], ids_sem)
                else:
                    @pl.when(i < N_INDEX - 1)
                    def _():
                        pltpu.async_copy(
                            ids_hbm.at[i + 1, pl.ds(0, CHUNK)],
                            ids_bufs[0], ids_sem)
                @pl.when(jnp.logical_or(i > 0, cc >= 2))
                def _():
                    pltpu.make_async_copy(
                        out_v, out_hbm.at[0, d, pl.ds(0, CHUNK)],
                        out_sem).wait()

                def gather_body(k, carry2):
                    base = k * (nl * UNROLL)
                    for u in range(UNROLL):
                        off = base + u * nl
                        idx = ids_v[pl.ds(off, nl)]
                        out_v[pl.ds(off, nl)] = plsc.load_gather(row_v, [idx])
                    return carry2

                lax.fori_loop(0, CHUNK // (nl * UNROLL), gather_body, 0)
                pltpu.async_copy(
                    out_v, out_hbm.at[i, d, pl.ds(cc * CHUNK, CHUNK)],
                    out_sem)
            @pl.when(i < N_INDEX - 1)
            def _():
                pltpu.async_copy(tab_hbm.at[i + 1, d, :], row_v, row_sem)
            return carry

        lax.fori_loop(0, N_INDEX, table_body, 0)
        for b in range(2):
            pltpu.make_async_copy(
                out_bufs[b], out_hbm.at[0, d, pl.ds(0, CHUNK)],
                out_sem).wait()

    return embed_kernel


def kernel(input_ids, tables):
    embed_kernel = _build()
    ids_t = input_ids.T
    tab_t = jnp.transpose(tables, (0, 2, 1))
    out_t = embed_kernel(ids_t, tab_t)
    return jnp.transpose(out_t, (2, 0, 1))
